# Initial kernel scaffold; baseline (speedup 1.0000x reference)
import functools

import jax
import jax.numpy as jnp
from jax import lax
from jax.experimental import pallas as pl
from jax.experimental.pallas import tpu as pltpu
from jax.experimental.pallas import tpu_sc as plsc

N = 10000
E = 320000
D = 128
H = 8
DH = 128

NC = 2
NS = 16
NW = NC * NS
EPW = E // NW
GB = 80
NGB = EPW // GB
GD = 40
NGD = EPW // GD
NPT = N // NS

_f32 = jnp.float32
_i32 = jnp.int32


def _vmesh():
    return plsc.VectorSubcoreMesh(core_axis_name="c", subcore_axis_name="s")


def _gather16(v, idx):
    return lax.gather(
        v,
        idx[:, None],
        dimension_numbers=lax.GatherDimensionNumbers(
            offset_dims=(), collapsed_slice_dims=(0,), start_index_map=(0,)),
        slice_sizes=(1,),
        mode=lax.GatherScatterMode.PROMISE_IN_BOUNDS,
    )


def _rot8(v):
    idx = lax.rem(lax.iota(_i32, 16) + 8, jnp.full((16,), 16, _i32))
    return _gather16(v, idx)


def _splat(v, lane):
    return _gather16(v, jnp.full((16,), lane, _i32))


def _sc_alpha_body(al_hbm, src_hbm, dst_hbm, z16_hbm, wraw_hbm, dpart_hbm,
                   srci, dsti, gs, gd, wv, dsh, sem):
    cid = lax.axis_index("c")
    sid = lax.axis_index("s")
    wid = cid * NS + sid
    pltpu.sync_copy(z16_hbm.at[pl.ds(sid * NPT, NPT), :],
                    dsh.at[pl.ds(sid * NPT, NPT), :])
    pltpu.sync_copy(src_hbm.at[wid], srci)
    pltpu.sync_copy(dst_hbm.at[wid], dsti)
    plsc.subcore_barrier()

    def grp(g, carry):
        c1 = pltpu.make_async_copy(al_hbm.at[srci.at[g]], gs, sem)
        c1.start()
        c2 = pltpu.make_async_copy(al_hbm.at[dsti.at[g]], gd, sem)
        c2.start()
        c1.wait()
        c2.wait()

        def ed(i, c):
            s = gs[i] + _rot8(gd[i])
            wv[i] = jnp.exp(jnp.maximum(s, 0.2 * s))
            return c
        lax.fori_loop(0, GB, ed, 0)
        pltpu.sync_copy(wv, dsh.at[dsti.at[g]], add=True)
        pltpu.sync_copy(wv, wraw_hbm.at[pl.ds(wid * EPW + g * GB, GB), :])
        return carry

    lax.fori_loop(0, NGB, grp, 0)
    plsc.subcore_barrier()
    pltpu.sync_copy(dsh.at[pl.ds(sid * NPT, NPT), :],
                    dpart_hbm.at[cid, pl.ds(sid * NPT, NPT), :])


_sc_alpha = functools.partial(
    pl.kernel,
    out_type=(jax.ShapeDtypeStruct((E, 16), _f32),
              jax.ShapeDtypeStruct((NC, N, 16), _f32)),
    mesh=_vmesh(),
    scratch_types=[
        pltpu.VMEM((NGB, GB), _i32),
        pltpu.VMEM((NGB, GB), _i32),
        pltpu.VMEM((GB, 16), _f32),
        pltpu.VMEM((GB, 16), _f32),
        pltpu.VMEM((GB, 16), _f32),
        pltpu.VMEM_SHARED((N, 16), _f32),
        pltpu.SemaphoreType.DMA,
    ],
)(_sc_alpha_body)


def _sc_msg_body(h_hbm, wraw_hbm, d0_hbm, d1_hbm, src_hbm, dst_hbm, z128_hbm,
                 opart_hbm, srci, dsti, hg, wg, d0g, d1g, wr, msg, osh, sem):
    cid = lax.axis_index("c")
    sid = lax.axis_index("s")
    wid = cid * NS + sid
    pltpu.sync_copy(z128_hbm.at[pl.ds(sid * NPT, NPT), :],
                    osh.at[pl.ds(sid * NPT, NPT), :])
    pltpu.sync_copy(src_hbm.at[wid], srci)
    pltpu.sync_copy(dst_hbm.at[wid], dsti)
    plsc.subcore_barrier()

    def grp(g, carry):
        base = wid * EPW + g * GD
        c1 = pltpu.make_async_copy(h_hbm.at[srci.at[g]], hg, sem)
        c1.start()
        c2 = pltpu.make_async_copy(wraw_hbm.at[pl.ds(base, GD), :], wg, sem)
        c2.start()
        c3 = pltpu.make_async_copy(d0_hbm.at[dsti.at[g]], d0g, sem)
        c3.start()
        c4 = pltpu.make_async_copy(d1_hbm.at[dsti.at[g]], d1g, sem)
        c4.start()
        c1.wait()
        c2.wait()
        c3.wait()
        c4.wait()

        def wprep(i, c):
            wr[i] = wg[i] / ((d0g[i] + d1g[i]) * 8.0)
            return c
        lax.fori_loop(0, GD, wprep, 0)

        def ed(i, c):
            wrow = wr[i]
            accs = [None] * 8
            for h in range(H):
                s = _splat(wrow, h)
                for j in range(8):
                    t = s * hg[i, pl.ds(h * 128 + j * 16, 16)]
                    accs[j] = t if h == 0 else accs[j] + t
            for j in range(8):
                msg[i, pl.ds(j * 16, 16)] = accs[j]
            return c
        lax.fori_loop(0, GD, ed, 0)
        pltpu.sync_copy(msg, osh.at[dsti.at[g]], add=True)
        return carry

    lax.fori_loop(0, NGD, grp, 0)
    plsc.subcore_barrier()
    pltpu.sync_copy(osh.at[pl.ds(sid * NPT, NPT), :],
                    opart_hbm.at[cid, pl.ds(sid * NPT, NPT), :])


_sc_msg = functools.partial(
    pl.kernel,
    out_type=jax.ShapeDtypeStruct((NC, N, 128), _f32),
    mesh=_vmesh(),
    scratch_types=[
        pltpu.VMEM((NGD, GD), _i32),
        pltpu.VMEM((NGD, GD), _i32),
        pltpu.VMEM((GD, 1024), _f32),
        pltpu.VMEM((GD, 16), _f32),
        pltpu.VMEM((GD, 16), _f32),
        pltpu.VMEM((GD, 16), _f32),
        pltpu.VMEM((GD, 16), _f32),
        pltpu.VMEM((GD, 128), _f32),
        pltpu.VMEM_SHARED((N, 128), _f32),
        pltpu.SemaphoreType.DMA,
    ],
)(_sc_msg_body)


_BM = 2000


def _tc_l1_body(x_ref, w_ref, p_ref, h_ref, al_ref):
    xb = x_ref[...]
    h_ref[...] = jnp.dot(xb, w_ref[...], preferred_element_type=_f32)
    al_ref[...] = jnp.dot(xb, p_ref[...], preferred_element_type=_f32)


def _tc_layer1(x, W, Pc):
    return pl.pallas_call(
        _tc_l1_body,
        grid=(N // _BM,),
        in_specs=[
            pl.BlockSpec((_BM, D), lambda i: (i, 0)),
            pl.BlockSpec((D, H * DH), lambda i: (0, 0)),
            pl.BlockSpec((D, 16), lambda i: (0, 0)),
        ],
        out_specs=[
            pl.BlockSpec((_BM, H * DH), lambda i: (i, 0)),
            pl.BlockSpec((_BM, 16), lambda i: (i, 0)),
        ],
        out_shape=[
            jax.ShapeDtypeStruct((N, H * DH), _f32),
            jax.ShapeDtypeStruct((N, 16), _f32),
        ],
    )(x, W, Pc)


def _tc_l2_body(p0_ref, p1_ref, b_ref, w_ref, p_ref, h_ref, al_ref):
    xb = jnp.maximum(p0_ref[...] + p1_ref[...] + b_ref[...], 0.0)
    h_ref[...] = jnp.dot(xb, w_ref[...], preferred_element_type=_f32)
    al_ref[...] = jnp.dot(xb, p_ref[...], preferred_element_type=_f32)


def _tc_layer2(p0, p1, b, W, Pc):
    return pl.pallas_call(
        _tc_l2_body,
        grid=(N // _BM,),
        in_specs=[
            pl.BlockSpec((_BM, D), lambda i: (i, 0)),
            pl.BlockSpec((_BM, D), lambda i: (i, 0)),
            pl.BlockSpec((1, D), lambda i: (0, 0)),
            pl.BlockSpec((D, H * DH), lambda i: (0, 0)),
            pl.BlockSpec((D, 16), lambda i: (0, 0)),
        ],
        out_specs=[
            pl.BlockSpec((_BM, H * DH), lambda i: (i, 0)),
            pl.BlockSpec((_BM, 16), lambda i: (i, 0)),
        ],
        out_shape=[
            jax.ShapeDtypeStruct((N, H * DH), _f32),
            jax.ShapeDtypeStruct((N, 16), _f32),
        ],
    )(p0, p1, b.reshape(1, D), W, Pc)


def _tc_final_body(p0_ref, p1_ref, b_ref, o_ref):
    o_ref[...] = p0_ref[...] + p1_ref[...] + b_ref[...]


def _tc_final(p0, p1, b):
    return pl.pallas_call(
        _tc_final_body,
        grid=(N // _BM,),
        in_specs=[
            pl.BlockSpec((_BM, D), lambda i: (i, 0)),
            pl.BlockSpec((_BM, D), lambda i: (i, 0)),
            pl.BlockSpec((1, D), lambda i: (0, 0)),
        ],
        out_specs=pl.BlockSpec((_BM, D), lambda i: (i, 0)),
        out_shape=jax.ShapeDtypeStruct((N, D), _f32),
    )(p0, p1, b.reshape(1, D))


def _pack_att(W, a_src, a_dst):
    Wr = W.reshape(D, H, DH)
    ps = jnp.einsum("ihd,hd->ih", Wr, a_src)
    pd = jnp.einsum("ihd,hd->ih", Wr, a_dst)
    return jnp.concatenate([ps, pd], axis=1)


def kernel(x, edge_index, W1, a_src1, a_dst1, b1, W2, a_src2, a_dst2, b2):
    src = edge_index[0].astype(_i32)
    dst = edge_index[1].astype(_i32)
    srcb = src.reshape(NW, NGB, GB)
    dstb = dst.reshape(NW, NGB, GB)
    srcd = src.reshape(NW, NGD, GD)
    dstd = dst.reshape(NW, NGD, GD)
    z16 = jnp.zeros((N, 16), _f32)
    z128 = jnp.zeros((N, 128), _f32)
    Pc1 = _pack_att(W1, a_src1, a_dst1)
    Pc2 = _pack_att(W2, a_src2, a_dst2)

    h1, al1 = _tc_layer1(x, W1, Pc1)
    wraw1, dp1 = _sc_alpha(al1, srcb, dstb, z16)
    op1 = _sc_msg(h1, wraw1, dp1[0], dp1[1], srcd, dstd, z128)

    h2, al2 = _tc_layer2(op1[0], op1[1], b1, W2, Pc2)
    wraw2, dp2 = _sc_alpha(al2, srcb, dstb, z16)
    op2 = _sc_msg(h2, wraw2, dp2[0], dp2[1], srcd, dstd, z128)

    return _tc_final(op2[0], op2[1], b2)


# speedup vs baseline: 13.3979x; 13.3979x over previous
import functools

import jax
import jax.numpy as jnp
from jax import lax
from jax.experimental import pallas as pl
from jax.experimental.pallas import tpu as pltpu
from jax.experimental.pallas import tpu_sc as plsc

N = 10000
E = 320000
D = 128
H = 8
DH = 128
DA = H * DH + 128

NC = 2
NS = 16
NW = NC * NS
EPW = E // NW

GA = 80
NGA = EPW // GA
GD = 40
NGD = EPW // GD
G3 = 80
NG3 = EPW // G3

NR = 4664
NP = 3
RP = NR + 8
HPT = 288
HREM = RP - NS * HPT

_f32 = jnp.float32
_i32 = jnp.int32


def _vmesh():
    return plsc.VectorSubcoreMesh(core_axis_name="c", subcore_axis_name="s")


def _gather16(v, idx):
    return lax.gather(
        v,
        idx[:, None],
        dimension_numbers=lax.GatherDimensionNumbers(
            offset_dims=(), collapsed_slice_dims=(0,), start_index_map=(0,)),
        slice_sizes=(1,),
        mode=lax.GatherScatterMode.PROMISE_IN_BOUNDS,
    )


def _rot8(v):
    idx = lax.rem(lax.iota(_i32, 16) + 8, jnp.full((16,), 16, _i32))
    return _gather16(v, idx)


def _splat(v, lane):
    return _gather16(v, jnp.full((16,), lane, _i32))


def _sc_wraw_body(al_hbm, src_hbm, dst_hbm, w_hbm,
                  srci, dsti, gs, gd, wv, sem):
    cid = lax.axis_index("c")
    sid = lax.axis_index("s")
    wid = cid * NS + sid
    pltpu.sync_copy(src_hbm.at[wid], srci)
    pltpu.sync_copy(dst_hbm.at[wid], dsti)

    zv = jnp.zeros((16,), _f32)

    def zrow(i, c):
        for jj in range(7):
            wv[i, pl.ds(16 + 16 * jj, 16)] = zv
        return c
    lax.fori_loop(0, GA, zrow, 0)

    def grp(g, carry):
        c1 = pltpu.make_async_copy(al_hbm.at[srci.at[g]], gs, sem)
        c1.start()
        c2 = pltpu.make_async_copy(al_hbm.at[dsti.at[g]], gd, sem)
        c2.start()
        c1.wait()
        c2.wait()

        def ed(i, c):
            s = gs[i, pl.ds(0, 16)] + _rot8(gd[i, pl.ds(0, 16)])
            wv[i, pl.ds(0, 16)] = jnp.exp(jnp.maximum(s, 0.2 * s))
            return c
        lax.fori_loop(0, GA, ed, 0)
        pltpu.sync_copy(wv, w_hbm.at[pl.ds(wid * EPW + g * GA, GA), :])
        return carry

    lax.fori_loop(0, NGA, grp, 0)


_sc_wraw = functools.partial(
    pl.kernel,
    out_type=jax.ShapeDtypeStruct((E, 128), _f32),
    mesh=_vmesh(),
    scratch_types=[
        pltpu.VMEM((NGA, GA), _i32),
        pltpu.VMEM((NGA, GA), _i32),
        pltpu.VMEM((GA, 128), _f32),
        pltpu.VMEM((GA, 128), _f32),
        pltpu.VMEM((GA, 128), _f32),
        pltpu.SemaphoreType.DMA,
    ],
)(_sc_wraw_body)


def _sc_msg_body(h_hbm, t_hbm, src_hbm, dst_hbm, m_hbm,
                 srci, dsti, hg, tg, msg, sem):
    cid = lax.axis_index("c")
    sid = lax.axis_index("s")
    wid = cid * NS + sid
    pltpu.sync_copy(src_hbm.at[wid], srci)
    pltpu.sync_copy(dst_hbm.at[wid], dsti)

    def grp(g, carry):
        c1 = pltpu.make_async_copy(h_hbm.at[srci.at[g]], hg, sem)
        c1.start()
        c2 = pltpu.make_async_copy(t_hbm.at[dsti.at[g]], tg, sem)
        c2.start()
        c1.wait()
        c2.wait()

        def ed(i, c):
            s = hg[i, pl.ds(H * DH, 16)]
            t = tg[i, pl.ds(0, 16)]
            u = s + t
            wr = jnp.exp(jnp.maximum(u, 0.2 * u)) * _rot8(u)
            accs = [None] * 8
            for h in range(H):
                sp = _splat(wr, h)
                for j in range(8):
                    tt = sp * hg[i, pl.ds(h * 128 + j * 16, 16)]
                    accs[j] = tt if h == 0 else accs[j] + tt
            for j in range(8):
                msg[i, pl.ds(j * 16, 16)] = accs[j]
            return c
        lax.fori_loop(0, GD, ed, 0)
        pltpu.sync_copy(msg, m_hbm.at[pl.ds(wid * EPW + g * GD, GD), :])
        return carry

    lax.fori_loop(0, NGD, grp, 0)


_sc_msg = functools.partial(
    pl.kernel,
    out_type=jax.ShapeDtypeStruct((E, 128), _f32),
    mesh=_vmesh(),
    scratch_types=[
        pltpu.VMEM((NGD, GD), _i32),
        pltpu.VMEM((NGD, GD), _i32),
        pltpu.VMEM((GD, DA), _f32),
        pltpu.VMEM((GD, 128), _f32),
        pltpu.VMEM((GD, 128), _f32),
        pltpu.SemaphoreType.DMA,
    ],
)(_sc_msg_body)


def _sc_scat_body(r_hbm, dst_hbm, z_hbm, opart_hbm,
                  dsti, dstt, rows, acc, sem):
    cid = lax.axis_index("c")
    sid = lax.axis_index("s")
    wid = cid * NS + sid
    pltpu.sync_copy(dst_hbm.at[wid], dsti)
    trap = jnp.full((16,), NR, _i32)

    for k in range(NP):
        def zc(kk, c):
            pltpu.sync_copy(z_hbm, acc.at[pl.ds(sid * HPT + kk * 8, 8), :])
            return c
        lax.fori_loop(0, HPT // 8, zc, 0)

        @pl.when(sid == 0)
        def _():
            def zc2(kk, c):
                pltpu.sync_copy(z_hbm, acc.at[pl.ds(NS * HPT + kk * 8, 8), :])
                return c
            lax.fori_loop(0, HREM // 8, zc2, 0)

        plsc.subcore_barrier()
        base = jnp.broadcast_to(jnp.int32(k * NR), (16,))

        def grp(g, carry):
            c1 = pltpu.make_async_copy(
                r_hbm.at[pl.ds(wid * EPW + g * G3, G3), :], rows, sem)
            c1.start()
            for kk in range(G3 // 16):
                v = dsti[g, pl.ds(kk * 16, 16)] - base
                m = (v >= 0) & (v < NR)
                dstt[pl.ds(kk * 16, 16)] = jnp.where(m, v, trap)
            c1.wait()
            pltpu.sync_copy(rows, acc.at[dstt], add=True)
            return carry

        lax.fori_loop(0, NG3, grp, 0)
        plsc.subcore_barrier()
        pltpu.sync_copy(acc.at[pl.ds(sid * HPT, HPT), :],
                        opart_hbm.at[k, cid, pl.ds(sid * HPT, HPT), :])

        @pl.when(sid == 0)
        def _():
            pltpu.sync_copy(acc.at[pl.ds(NS * HPT, HREM), :],
                            opart_hbm.at[k, cid, pl.ds(NS * HPT, HREM), :])

        plsc.subcore_barrier()


_sc_scat = functools.partial(
    pl.kernel,
    out_type=jax.ShapeDtypeStruct((NP, NC, RP, 128), _f32),
    mesh=_vmesh(),
    scratch_types=[
        pltpu.VMEM((NG3, G3), _i32),
        pltpu.VMEM((G3,), _i32),
        pltpu.VMEM((G3, 128), _f32),
        pltpu.VMEM_SHARED((RP, 128), _f32),
        pltpu.SemaphoreType.DMA,
    ],
)(_sc_scat_body)


def _segment_sum_parts(rows, dsts, z8):
    p = _sc_scat(rows, dsts, z8)
    s0 = jnp.concatenate(
        [p[0, 0, :NR], p[1, 0, :NR], p[2, 0, :N - 2 * NR]], axis=0)
    s1 = jnp.concatenate(
        [p[0, 1, :NR], p[1, 1, :NR], p[2, 1, :N - 2 * NR]], axis=0)
    return s0, s1


_BM = 2000


def _tc_l1_body(x_ref, w_ref, p_ref, h_ref, al_ref):
    xb = x_ref[...]
    h_ref[...] = jnp.dot(xb, w_ref[...], preferred_element_type=_f32)
    al_ref[...] = jnp.dot(xb, p_ref[...], preferred_element_type=_f32)


def _tc_layer1(x, Wcat, Pc):
    return pl.pallas_call(
        _tc_l1_body,
        grid=(N // _BM,),
        in_specs=[
            pl.BlockSpec((_BM, D), lambda i: (i, 0)),
            pl.BlockSpec((D, DA), lambda i: (0, 0)),
            pl.BlockSpec((D, 128), lambda i: (0, 0)),
        ],
        out_specs=[
            pl.BlockSpec((_BM, DA), lambda i: (i, 0)),
            pl.BlockSpec((_BM, 128), lambda i: (i, 0)),
        ],
        out_shape=[
            jax.ShapeDtypeStruct((N, DA), _f32),
            jax.ShapeDtypeStruct((N, 128), _f32),
        ],
    )(x, Wcat, Pc)


def _tc_l2_body(p0_ref, p1_ref, b_ref, w_ref, pc_ref, h_ref, al_ref):
    xb = jnp.maximum(p0_ref[...] + p1_ref[...] + b_ref[...], 0.0)
    h_ref[...] = jnp.dot(xb, w_ref[...], preferred_element_type=_f32)
    al_ref[...] = jnp.dot(xb, pc_ref[...], preferred_element_type=_f32)


def _tc_layer2(p0, p1, b, Wcat, Pc):
    return pl.pallas_call(
        _tc_l2_body,
        grid=(N // _BM,),
        in_specs=[
            pl.BlockSpec((_BM, D), lambda i: (i, 0)),
            pl.BlockSpec((_BM, D), lambda i: (i, 0)),
            pl.BlockSpec((1, D), lambda i: (0, 0)),
            pl.BlockSpec((D, DA), lambda i: (0, 0)),
            pl.BlockSpec((D, 128), lambda i: (0, 0)),
        ],
        out_specs=[
            pl.BlockSpec((_BM, DA), lambda i: (i, 0)),
            pl.BlockSpec((_BM, 128), lambda i: (i, 0)),
        ],
        out_shape=[
            jax.ShapeDtypeStruct((N, DA), _f32),
            jax.ShapeDtypeStruct((N, 128), _f32),
        ],
    )(p0, p1, b.reshape(1, D), Wcat, Pc)


def _tc_tdst_body(d0_ref, d1_ref, al_ref, t_ref):
    den = d0_ref[...] + d1_ref[...]
    rden = 1.0 / (8.0 * den[:, 0:8])
    adst = al_ref[...][:, 8:16]
    t_ref[...] = jnp.concatenate(
        [adst, rden, jnp.zeros((d0_ref.shape[0], 112), _f32)], axis=1)


def _tc_tdst(d0, d1, al):
    return pl.pallas_call(
        _tc_tdst_body,
        grid=(N // _BM,),
        in_specs=[
            pl.BlockSpec((_BM, 128), lambda i: (i, 0)),
            pl.BlockSpec((_BM, 128), lambda i: (i, 0)),
            pl.BlockSpec((_BM, 128), lambda i: (i, 0)),
        ],
        out_specs=pl.BlockSpec((_BM, 128), lambda i: (i, 0)),
        out_shape=jax.ShapeDtypeStruct((N, 128), _f32),
    )(d0, d1, al)


def _tc_final_body(p0_ref, p1_ref, b_ref, o_ref):
    o_ref[...] = p0_ref[...] + p1_ref[...] + b_ref[...]


def _tc_final(p0, p1, b):
    return pl.pallas_call(
        _tc_final_body,
        grid=(N // _BM,),
        in_specs=[
            pl.BlockSpec((_BM, D), lambda i: (i, 0)),
            pl.BlockSpec((_BM, D), lambda i: (i, 0)),
            pl.BlockSpec((1, D), lambda i: (0, 0)),
        ],
        out_specs=pl.BlockSpec((_BM, D), lambda i: (i, 0)),
        out_shape=jax.ShapeDtypeStruct((N, D), _f32),
    )(p0, p1, b.reshape(1, D))


def _pack_weights(W, a_src, a_dst):
    Wr = W.reshape(D, H, DH)
    ps = jnp.einsum("ihd,hd->ih", Wr, a_src)
    pd = jnp.einsum("ihd,hd->ih", Wr, a_dst)
    Wcat = jnp.concatenate([W, ps, jnp.zeros((D, 120), _f32)], axis=1)
    Pc = jnp.concatenate([ps, pd, jnp.zeros((D, 112), _f32)], axis=1)
    return Wcat, Pc


def _gat_layer(h, al, srca, dsta, srcd, dstd, dsts, z8):
    w = _sc_wraw(al, srca, dsta)
    d0, d1 = _segment_sum_parts(w, dsts, z8)
    t = _tc_tdst(d0, d1, al)
    m = _sc_msg(h, t, srcd, dstd)
    return _segment_sum_parts(m, dsts, z8)


def kernel(x, edge_index, W1, a_src1, a_dst1, b1, W2, a_src2, a_dst2, b2):
    src = edge_index[0].astype(_i32)
    dst = edge_index[1].astype(_i32)
    srca = src.reshape(NW, NGA, GA)
    dsta = dst.reshape(NW, NGA, GA)
    srcd = src.reshape(NW, NGD, GD)
    dstd = dst.reshape(NW, NGD, GD)
    dsts = dst.reshape(NW, NG3, G3)
    z8 = jnp.zeros((8, 128), _f32)
    Wcat1, Pc1 = _pack_weights(W1, a_src1, a_dst1)
    Wcat2, Pc2 = _pack_weights(W2, a_src2, a_dst2)

    h1, al1 = _tc_layer1(x, Wcat1, Pc1)
    o10, o11 = _gat_layer(h1, al1, srca, dsta, srcd, dstd, dsts, z8)

    h2, al2 = _tc_layer2(o10, o11, b1, Wcat2, Pc2)
    o20, o21 = _gat_layer(h2, al2, srca, dsta, srcd, dstd, dsts, z8)

    return _tc_final(o20, o21, b2)


# speedup vs baseline: 19.7220x; 1.4720x over previous
import functools

import jax
import jax.numpy as jnp
from jax import lax
from jax.experimental import pallas as pl
from jax.experimental.pallas import tpu as pltpu
from jax.experimental.pallas import tpu_sc as plsc

N = 10000
E = 320000
D = 128
H = 8
DH = 128
DA = H * DH + 128

NC = 2
NS = 16
NW = NC * NS
EPW = E // NW

GA = 80
NGA = EPW // GA
GD = 40
NGD = EPW // GD
G3 = 80
NG3 = EPW // G3

NR = 4664
NP = 3
RP = NR + 8
HPT = 288
HREM = RP - NS * HPT

_f32 = jnp.float32
_i32 = jnp.int32


def _vmesh():
    return plsc.VectorSubcoreMesh(core_axis_name="c", subcore_axis_name="s")


def _gather16(v, idx):
    return lax.gather(
        v,
        idx[:, None],
        dimension_numbers=lax.GatherDimensionNumbers(
            offset_dims=(), collapsed_slice_dims=(0,), start_index_map=(0,)),
        slice_sizes=(1,),
        mode=lax.GatherScatterMode.PROMISE_IN_BOUNDS,
    )


def _rot8(v):
    idx = lax.rem(lax.iota(_i32, 16) + 8, jnp.full((16,), 16, _i32))
    return _gather16(v, idx)


def _splat(v, lane):
    return _gather16(v, jnp.full((16,), lane, _i32))


def _sc_wraw_body(al_hbm, src_hbm, dst_hbm, w_hbm,
                  srci, dsti, gs, gd, wv, sem):
    cid = lax.axis_index("c")
    sid = lax.axis_index("s")
    wid = cid * NS + sid
    pltpu.sync_copy(src_hbm.at[wid], srci)
    pltpu.sync_copy(dst_hbm.at[wid], dsti)

    zv = jnp.zeros((16,), _f32)

    def zrow(i, c):
        for jj in range(7):
            wv[i, pl.ds(16 + 16 * jj, 16)] = zv
        return c
    lax.fori_loop(0, GA, zrow, 0)

    def grp(g, carry):
        c1 = pltpu.make_async_copy(al_hbm.at[srci.at[g]], gs, sem)
        c1.start()
        c2 = pltpu.make_async_copy(al_hbm.at[dsti.at[g]], gd, sem)
        c2.start()
        c1.wait()
        c2.wait()

        def ed(i, c):
            s = gs[i, pl.ds(0, 16)] + _rot8(gd[i, pl.ds(0, 16)])
            wv[i, pl.ds(0, 16)] = jnp.exp(jnp.maximum(s, 0.2 * s))
            return c
        lax.fori_loop(0, GA, ed, 0)
        pltpu.sync_copy(wv, w_hbm.at[pl.ds(wid * EPW + g * GA, GA), :])
        return carry

    lax.fori_loop(0, NGA, grp, 0)


_sc_wraw = functools.partial(
    pl.kernel,
    out_type=jax.ShapeDtypeStruct((E, 128), _f32),
    mesh=_vmesh(),
    scratch_types=[
        pltpu.VMEM((NGA, GA), _i32),
        pltpu.VMEM((NGA, GA), _i32),
        pltpu.VMEM((GA, 128), _f32),
        pltpu.VMEM((GA, 128), _f32),
        pltpu.VMEM((GA, 128), _f32),
        pltpu.SemaphoreType.DMA,
    ],
)(_sc_wraw_body)


NCH = 5
CG = NGD // NCH


def _msg_compute(hg, tg, msg):
    def ed(i, c):
        s = hg[i, pl.ds(H * DH, 16)]
        t = tg[i, pl.ds(0, 16)]
        u = s + t
        wr = jnp.exp(jnp.maximum(u, 0.2 * u)) * _rot8(u)
        accs = [None] * 8
        for h in range(H):
            sp = _splat(wr, h)
            for j in range(8):
                tt = sp * hg[i, pl.ds(h * 128 + j * 16, 16)]
                accs[j] = tt if h == 0 else accs[j] + tt
        for j in range(8):
            msg[i, pl.ds(j * 16, 16)] = accs[j]
        return c
    lax.fori_loop(0, GD, ed, 0)


def _sc_msg_body(h_hbm, t_hbm, src_hbm, dst_hbm, m_hbm,
                 srci, dsti, hgA, tgA, msgA, hgB, tgB, msgB,
                 lsA, lsB, ssem):
    cid = lax.axis_index("c")
    sid = lax.axis_index("s")
    wid = cid * NS + sid

    def start_ld(g, hg, tg, ls):
        pltpu.make_async_copy(h_hbm.at[srci.at[g]], hg, ls).start()
        pltpu.make_async_copy(t_hbm.at[dsti.at[g]], tg, ls).start()

    def wait_ld(g, hg, tg, ls):
        pltpu.make_async_copy(h_hbm.at[srci.at[g]], hg, ls).wait()
        pltpu.make_async_copy(t_hbm.at[dsti.at[g]], tg, ls).wait()

    for o in range(NCH):
        pltpu.sync_copy(src_hbm.at[wid, o], srci)
        pltpu.sync_copy(dst_hbm.at[wid, o], dsti)
        gbase = wid * EPW + o * CG * GD
        start_ld(0, hgA, tgA, lsA)

        def pair(p, c):
            g0 = 2 * p
            g1 = g0 + 1
            start_ld(g1, hgB, tgB, lsB)

            @pl.when(p > 0)
            def _():
                pltpu.make_async_copy(
                    msgA, m_hbm.at[pl.ds(gbase, GD), :], ssem).wait()
                pltpu.make_async_copy(
                    msgB, m_hbm.at[pl.ds(gbase, GD), :], ssem).wait()

            wait_ld(g0, hgA, tgA, lsA)
            _msg_compute(hgA, tgA, msgA)
            pltpu.make_async_copy(
                msgA, m_hbm.at[pl.ds(gbase + g0 * GD, GD), :], ssem).start()

            @pl.when(p < CG // 2 - 1)
            def _():
                start_ld(g0 + 2, hgA, tgA, lsA)

            wait_ld(g1, hgB, tgB, lsB)
            _msg_compute(hgB, tgB, msgB)
            pltpu.make_async_copy(
                msgB, m_hbm.at[pl.ds(gbase + g1 * GD, GD), :], ssem).start()
            return c

        lax.fori_loop(0, CG // 2, pair, 0)
        pltpu.make_async_copy(msgA, m_hbm.at[pl.ds(gbase, GD), :], ssem).wait()
        pltpu.make_async_copy(msgB, m_hbm.at[pl.ds(gbase, GD), :], ssem).wait()


_sc_msg = functools.partial(
    pl.kernel,
    out_type=jax.ShapeDtypeStruct((E, 128), _f32),
    mesh=_vmesh(),
    scratch_types=[
        pltpu.VMEM((CG, GD), _i32),
        pltpu.VMEM((CG, GD), _i32),
        pltpu.VMEM((GD, DA), _f32),
        pltpu.VMEM((GD, 128), _f32),
        pltpu.VMEM((GD, 128), _f32),
        pltpu.VMEM((GD, DA), _f32),
        pltpu.VMEM((GD, 128), _f32),
        pltpu.VMEM((GD, 128), _f32),
        pltpu.SemaphoreType.DMA,
        pltpu.SemaphoreType.DMA,
        pltpu.SemaphoreType.DMA,
    ],
)(_sc_msg_body)


def _sc_scat_body(r_hbm, dst_hbm, z_hbm, opart_hbm,
                  dsti, dsttA, dsttB, rowsA, rowsB, acc,
                  lsA, lsB, ssA, ssB):
    cid = lax.axis_index("c")
    sid = lax.axis_index("s")
    wid = cid * NS + sid
    pltpu.sync_copy(dst_hbm.at[wid], dsti)
    trap = jnp.full((16,), NR, _i32)

    def ld(g, rows, ls):
        return pltpu.make_async_copy(
            r_hbm.at[pl.ds(wid * EPW + g * G3, G3), :], rows, ls)

    def remap(g, dstt, base):
        for kk in range(G3 // 16):
            v = dsti[g, pl.ds(kk * 16, 16)] - base
            m = (v >= 0) & (v < NR)
            dstt[pl.ds(kk * 16, 16)] = jnp.where(m, v, trap)

    for k in range(NP):
        pltpu.sync_copy(z_hbm, acc.at[pl.ds(sid * HPT, HPT), :])

        @pl.when(sid == 0)
        def _():
            pltpu.sync_copy(z_hbm.at[pl.ds(0, HREM), :],
                            acc.at[pl.ds(NS * HPT, HREM), :])

        plsc.subcore_barrier()
        base = jnp.broadcast_to(jnp.int32(k * NR), (16,))
        ld(0, rowsA, lsA).start()

        def pair(p, c):
            gA = 2 * p
            gB = gA + 1

            @pl.when(p > 0)
            def _():
                pltpu.make_async_copy(rowsB, acc.at[dsttB], ssB).wait()

            @pl.when(gB < NG3)
            def _():
                ld(gB, rowsB, lsB).start()

            ld(gA, rowsA, lsA).wait()
            remap(gA, dsttA, base)
            pltpu.make_async_copy(rowsA, acc.at[dsttA], ssA).start(add=True)

            @pl.when(gB < NG3)
            def _():
                ld(gB, rowsB, lsB).wait()
                remap(gB, dsttB, base)
                pltpu.make_async_copy(rowsB, acc.at[dsttB], ssB).start(
                    add=True)

            @pl.when(p < (NG3 + 1) // 2 - 1)
            def _():
                pltpu.make_async_copy(rowsA, acc.at[dsttA], ssA).wait()
                ld(gA + 2, rowsA, lsA).start()
            return c

        lax.fori_loop(0, (NG3 + 1) // 2, pair, 0)
        pltpu.make_async_copy(rowsA, acc.at[dsttA], ssA).wait()
        plsc.subcore_barrier()
        pltpu.sync_copy(acc.at[pl.ds(sid * HPT, HPT), :],
                        opart_hbm.at[k, cid, pl.ds(sid * HPT, HPT), :])

        @pl.when(sid == 0)
        def _():
            pltpu.sync_copy(acc.at[pl.ds(NS * HPT, HREM), :],
                            opart_hbm.at[k, cid, pl.ds(NS * HPT, HREM), :])

        plsc.subcore_barrier()


_sc_scat = functools.partial(
    pl.kernel,
    out_type=jax.ShapeDtypeStruct((NP, NC, RP, 128), _f32),
    mesh=_vmesh(),
    scratch_types=[
        pltpu.VMEM((NG3, G3), _i32),
        pltpu.VMEM((G3,), _i32),
        pltpu.VMEM((G3,), _i32),
        pltpu.VMEM((G3, 128), _f32),
        pltpu.VMEM((G3, 128), _f32),
        pltpu.VMEM_SHARED((RP, 128), _f32),
        pltpu.SemaphoreType.DMA,
        pltpu.SemaphoreType.DMA,
        pltpu.SemaphoreType.DMA,
        pltpu.SemaphoreType.DMA,
    ],
)(_sc_scat_body)


def _segment_sum_parts(rows, dsts, z8):
    p = _sc_scat(rows, dsts, z8)
    s0 = jnp.concatenate(
        [p[0, 0, :NR], p[1, 0, :NR], p[2, 0, :N - 2 * NR]], axis=0)
    s1 = jnp.concatenate(
        [p[0, 1, :NR], p[1, 1, :NR], p[2, 1, :N - 2 * NR]], axis=0)
    return s0, s1


_BM = 2000


def _tc_l1_body(x_ref, w_ref, p_ref, h_ref, al_ref):
    xb = x_ref[...]
    h_ref[...] = jnp.dot(xb, w_ref[...], preferred_element_type=_f32)
    al_ref[...] = jnp.dot(xb, p_ref[...], preferred_element_type=_f32)


def _tc_layer1(x, Wcat, Pc):
    return pl.pallas_call(
        _tc_l1_body,
        grid=(N // _BM,),
        in_specs=[
            pl.BlockSpec((_BM, D), lambda i: (i, 0)),
            pl.BlockSpec((D, DA), lambda i: (0, 0)),
            pl.BlockSpec((D, 128), lambda i: (0, 0)),
        ],
        out_specs=[
            pl.BlockSpec((_BM, DA), lambda i: (i, 0)),
            pl.BlockSpec((_BM, 128), lambda i: (i, 0)),
        ],
        out_shape=[
            jax.ShapeDtypeStruct((N, DA), _f32),
            jax.ShapeDtypeStruct((N, 128), _f32),
        ],
    )(x, Wcat, Pc)


def _tc_l2_body(p0_ref, p1_ref, b_ref, w_ref, pc_ref, h_ref, al_ref):
    xb = jnp.maximum(p0_ref[...] + p1_ref[...] + b_ref[...], 0.0)
    h_ref[...] = jnp.dot(xb, w_ref[...], preferred_element_type=_f32)
    al_ref[...] = jnp.dot(xb, pc_ref[...], preferred_element_type=_f32)


def _tc_layer2(p0, p1, b, Wcat, Pc):
    return pl.pallas_call(
        _tc_l2_body,
        grid=(N // _BM,),
        in_specs=[
            pl.BlockSpec((_BM, D), lambda i: (i, 0)),
            pl.BlockSpec((_BM, D), lambda i: (i, 0)),
            pl.BlockSpec((1, D), lambda i: (0, 0)),
            pl.BlockSpec((D, DA), lambda i: (0, 0)),
            pl.BlockSpec((D, 128), lambda i: (0, 0)),
        ],
        out_specs=[
            pl.BlockSpec((_BM, DA), lambda i: (i, 0)),
            pl.BlockSpec((_BM, 128), lambda i: (i, 0)),
        ],
        out_shape=[
            jax.ShapeDtypeStruct((N, DA), _f32),
            jax.ShapeDtypeStruct((N, 128), _f32),
        ],
    )(p0, p1, b.reshape(1, D), Wcat, Pc)


def _tc_tdst_body(d0_ref, d1_ref, al_ref, t_ref):
    den = d0_ref[...] + d1_ref[...]
    rden = 1.0 / (8.0 * den[:, 0:8])
    adst = al_ref[...][:, 8:16]
    t_ref[...] = jnp.concatenate(
        [adst, rden, jnp.zeros((d0_ref.shape[0], 112), _f32)], axis=1)


def _tc_tdst(d0, d1, al):
    return pl.pallas_call(
        _tc_tdst_body,
        grid=(N // _BM,),
        in_specs=[
            pl.BlockSpec((_BM, 128), lambda i: (i, 0)),
            pl.BlockSpec((_BM, 128), lambda i: (i, 0)),
            pl.BlockSpec((_BM, 128), lambda i: (i, 0)),
        ],
        out_specs=pl.BlockSpec((_BM, 128), lambda i: (i, 0)),
        out_shape=jax.ShapeDtypeStruct((N, 128), _f32),
    )(d0, d1, al)


def _tc_final_body(p0_ref, p1_ref, b_ref, o_ref):
    o_ref[...] = p0_ref[...] + p1_ref[...] + b_ref[...]


def _tc_final(p0, p1, b):
    return pl.pallas_call(
        _tc_final_body,
        grid=(N // _BM,),
        in_specs=[
            pl.BlockSpec((_BM, D), lambda i: (i, 0)),
            pl.BlockSpec((_BM, D), lambda i: (i, 0)),
            pl.BlockSpec((1, D), lambda i: (0, 0)),
        ],
        out_specs=pl.BlockSpec((_BM, D), lambda i: (i, 0)),
        out_shape=jax.ShapeDtypeStruct((N, D), _f32),
    )(p0, p1, b.reshape(1, D))


def _pack_weights(W, a_src, a_dst):
    Wr = W.reshape(D, H, DH)
    ps = jnp.einsum("ihd,hd->ih", Wr, a_src)
    pd = jnp.einsum("ihd,hd->ih", Wr, a_dst)
    Wcat = jnp.concatenate([W, ps, jnp.zeros((D, 120), _f32)], axis=1)
    Pc = jnp.concatenate([ps, pd, jnp.zeros((D, 112), _f32)], axis=1)
    return Wcat, Pc


def _gat_layer(h, al, srca, dsta, srcd, dstd, dsts, z8):
    w = _sc_wraw(al, srca, dsta)
    d0, d1 = _segment_sum_parts(w, dsts, z8)
    t = _tc_tdst(d0, d1, al)
    m = _sc_msg(h, t, srcd, dstd)
    return _segment_sum_parts(m, dsts, z8)


def kernel(x, edge_index, W1, a_src1, a_dst1, b1, W2, a_src2, a_dst2, b2):
    src = edge_index[0].astype(_i32)
    dst = edge_index[1].astype(_i32)
    srca = src.reshape(NW, NGA, GA)
    dsta = dst.reshape(NW, NGA, GA)
    srcd = src.reshape(NW, NCH, CG, GD)
    dstd = dst.reshape(NW, NCH, CG, GD)
    dsts = dst.reshape(NW, NG3, G3)
    z8 = jnp.zeros((HPT, 128), _f32)
    Wcat1, Pc1 = _pack_weights(W1, a_src1, a_dst1)
    Wcat2, Pc2 = _pack_weights(W2, a_src2, a_dst2)

    h1, al1 = _tc_layer1(x, Wcat1, Pc1)
    o10, o11 = _gat_layer(h1, al1, srca, dsta, srcd, dstd, dsts, z8)

    h2, al2 = _tc_layer2(o10, o11, b1, Wcat2, Pc2)
    o20, o21 = _gat_layer(h2, al2, srca, dsta, srcd, dstd, dsts, z8)

    return _tc_final(o20, o21, b2)


# speedup vs baseline: 23.7783x; 1.2057x over previous
import functools

import jax
import jax.numpy as jnp
from jax import lax
from jax.experimental import pallas as pl
from jax.experimental.pallas import tpu as pltpu
from jax.experimental.pallas import tpu_sc as plsc

N = 10000
E = 320000
D = 128
H = 8
DH = 128
DA = H * DH + 128

NC = 2
NS = 16
NW = NC * NS
EPW = E // NW

GA = 80
NGA = EPW // GA
GD = 40
NGD = EPW // GD
G3 = 80
NG3 = EPW // G3

NR = 4664
NP = 3
RP = NR + 8
HPT = 288
HREM = RP - NS * HPT

_f32 = jnp.float32
_i32 = jnp.int32


def _vmesh():
    return plsc.VectorSubcoreMesh(core_axis_name="c", subcore_axis_name="s")


def _gather16(v, idx):
    return lax.gather(
        v,
        idx[:, None],
        dimension_numbers=lax.GatherDimensionNumbers(
            offset_dims=(), collapsed_slice_dims=(0,), start_index_map=(0,)),
        slice_sizes=(1,),
        mode=lax.GatherScatterMode.PROMISE_IN_BOUNDS,
    )


def _rot8(v):
    idx = lax.rem(lax.iota(_i32, 16) + 8, jnp.full((16,), 16, _i32))
    return _gather16(v, idx)


def _splat(v, lane):
    return _gather16(v, jnp.full((16,), lane, _i32))


def _sc_wraw_body(al_hbm, src_hbm, dst_hbm, w_hbm,
                  srci, dsti, gs, gd, wv, sem):
    cid = lax.axis_index("c")
    sid = lax.axis_index("s")
    wid = cid * NS + sid
    pltpu.sync_copy(src_hbm.at[wid], srci)
    pltpu.sync_copy(dst_hbm.at[wid], dsti)

    def grp(g, carry):
        c1 = pltpu.make_async_copy(al_hbm.at[srci.at[g]], gs, sem)
        c1.start()
        c2 = pltpu.make_async_copy(al_hbm.at[dsti.at[g]], gd, sem)
        c2.start()
        c1.wait()
        c2.wait()

        def blk(kb, c):
            lane = lax.iota(_i32, 16)
            lmask = jnp.minimum(jnp.maximum(8 - lane, 0), 1).astype(_f32)
            d16 = dsti[g, pl.ds(kb * 16, 16)]
            for ii in range(16):
                i = kb * 16 + ii
                slot = lax.bitwise_and(_splat(d16, ii),
                                       jnp.full((16,), 7, _i32))
                s = gs[i, pl.ds(0, 16)] + _rot8(gd[i, pl.ds(0, 16)])
                w = jnp.exp(jnp.maximum(s, 0.2 * s)) * lmask
                for jj in range(8):
                    eq = (1 - jnp.minimum(jnp.abs(slot - jj), 1)
                          ).astype(_f32)
                    wv[i, pl.ds(jj * 16, 16)] = w * eq
            return c
        lax.fori_loop(0, GA // 16, blk, 0)
        pltpu.sync_copy(wv, w_hbm.at[pl.ds(wid * EPW + g * GA, GA), :])
        return carry

    lax.fori_loop(0, NGA, grp, 0)


_sc_wraw = functools.partial(
    pl.kernel,
    out_type=jax.ShapeDtypeStruct((E, 128), _f32),
    mesh=_vmesh(),
    scratch_types=[
        pltpu.VMEM((NGA, GA), _i32),
        pltpu.VMEM((NGA, GA), _i32),
        pltpu.VMEM((GA, 128), _f32),
        pltpu.VMEM((GA, 128), _f32),
        pltpu.VMEM((GA, 128), _f32),
        pltpu.SemaphoreType.DMA,
    ],
)(_sc_wraw_body)


RP1 = 1280
HPT1 = RP1 // NS


def _sc_scat1_body(r_hbm, dst_hbm, z_hbm, opart_hbm,
                   dsti, dsttA, dsttB, rowsA, rowsB, acc,
                   lsA, lsB, ssA, ssB):
    cid = lax.axis_index("c")
    sid = lax.axis_index("s")
    wid = cid * NS + sid
    pltpu.sync_copy(dst_hbm.at[wid], dsti)
    pltpu.sync_copy(z_hbm.at[pl.ds(0, HPT1), :],
                    acc.at[pl.ds(sid * HPT1, HPT1), :])
    plsc.subcore_barrier()

    def ld(g, rows, ls):
        return pltpu.make_async_copy(
            r_hbm.at[pl.ds(wid * EPW + g * G3, G3), :], rows, ls)

    def remap(g, dstt):
        for kk in range(G3 // 16):
            dstt[pl.ds(kk * 16, 16)] = lax.shift_right_logical(
                dsti[g, pl.ds(kk * 16, 16)], 3)

    ld(0, rowsA, lsA).start()

    def pair(p, c):
        gA = 2 * p
        gB = gA + 1

        @pl.when(p > 0)
        def _():
            pltpu.make_async_copy(rowsB, acc.at[dsttB], ssB).wait()

        @pl.when(gB < NG3)
        def _():
            ld(gB, rowsB, lsB).start()

        ld(gA, rowsA, lsA).wait()
        remap(gA, dsttA)
        pltpu.make_async_copy(rowsA, acc.at[dsttA], ssA).start(add=True)

        @pl.when(gB < NG3)
        def _():
            ld(gB, rowsB, lsB).wait()
            remap(gB, dsttB)
            pltpu.make_async_copy(rowsB, acc.at[dsttB], ssB).start(add=True)

        @pl.when(p < (NG3 + 1) // 2 - 1)
        def _():
            pltpu.make_async_copy(rowsA, acc.at[dsttA], ssA).wait()
            ld(gA + 2, rowsA, lsA).start()
        return c

    lax.fori_loop(0, (NG3 + 1) // 2, pair, 0)
    pltpu.make_async_copy(rowsA, acc.at[dsttA], ssA).wait()
    plsc.subcore_barrier()
    pltpu.sync_copy(acc.at[pl.ds(sid * HPT1, HPT1), :],
                    opart_hbm.at[cid, pl.ds(sid * HPT1, HPT1), :])


_sc_scat1 = functools.partial(
    pl.kernel,
    out_type=jax.ShapeDtypeStruct((NC, RP1, 128), _f32),
    mesh=_vmesh(),
    scratch_types=[
        pltpu.VMEM((NG3, G3), _i32),
        pltpu.VMEM((G3,), _i32),
        pltpu.VMEM((G3,), _i32),
        pltpu.VMEM((G3, 128), _f32),
        pltpu.VMEM((G3, 128), _f32),
        pltpu.VMEM_SHARED((RP1, 128), _f32),
        pltpu.SemaphoreType.DMA,
        pltpu.SemaphoreType.DMA,
        pltpu.SemaphoreType.DMA,
        pltpu.SemaphoreType.DMA,
    ],
)(_sc_scat1_body)


NCH = 5
CG = NGD // NCH


def _msg_compute(hg, tg, msg):
    def ed(i, c):
        s = hg[i, pl.ds(H * DH, 16)]
        t = tg[i, pl.ds(0, 16)]
        u = s + t
        wr = jnp.exp(jnp.maximum(u, 0.2 * u)) * _rot8(u)
        accs = [None] * 8
        for h in range(H):
            sp = _splat(wr, h)
            for j in range(8):
                tt = sp * hg[i, pl.ds(h * 128 + j * 16, 16)]
                accs[j] = tt if h == 0 else accs[j] + tt
        for j in range(8):
            msg[i, pl.ds(j * 16, 16)] = accs[j]
        return c
    lax.fori_loop(0, GD, ed, 0)


def _sc_msg_body(h_hbm, t_hbm, src_hbm, dst_hbm, m_hbm,
                 srci, dsti, hgA, tgA, msgA, hgB, tgB, msgB,
                 lsA, lsB, ssem):
    cid = lax.axis_index("c")
    sid = lax.axis_index("s")
    wid = cid * NS + sid

    def start_ld(g, hg, tg, ls):
        pltpu.make_async_copy(h_hbm.at[srci.at[g]], hg, ls).start()
        pltpu.make_async_copy(t_hbm.at[dsti.at[g]], tg, ls).start()

    def wait_ld(g, hg, tg, ls):
        pltpu.make_async_copy(h_hbm.at[srci.at[g]], hg, ls).wait()
        pltpu.make_async_copy(t_hbm.at[dsti.at[g]], tg, ls).wait()

    for o in range(NCH):
        pltpu.sync_copy(src_hbm.at[wid, o], srci)
        pltpu.sync_copy(dst_hbm.at[wid, o], dsti)
        gbase = wid * EPW + o * CG * GD
        start_ld(0, hgA, tgA, lsA)

        def pair(p, c):
            g0 = 2 * p
            g1 = g0 + 1
            start_ld(g1, hgB, tgB, lsB)

            @pl.when(p > 0)
            def _():
                pltpu.make_async_copy(
                    msgA, m_hbm.at[pl.ds(gbase, GD), :], ssem).wait()
                pltpu.make_async_copy(
                    msgB, m_hbm.at[pl.ds(gbase, GD), :], ssem).wait()

            wait_ld(g0, hgA, tgA, lsA)
            _msg_compute(hgA, tgA, msgA)
            pltpu.make_async_copy(
                msgA, m_hbm.at[pl.ds(gbase + g0 * GD, GD), :], ssem).start()

            @pl.when(p < CG // 2 - 1)
            def _():
                start_ld(g0 + 2, hgA, tgA, lsA)

            wait_ld(g1, hgB, tgB, lsB)
            _msg_compute(hgB, tgB, msgB)
            pltpu.make_async_copy(
                msgB, m_hbm.at[pl.ds(gbase + g1 * GD, GD), :], ssem).start()
            return c

        lax.fori_loop(0, CG // 2, pair, 0)
        pltpu.make_async_copy(msgA, m_hbm.at[pl.ds(gbase, GD), :], ssem).wait()
        pltpu.make_async_copy(msgB, m_hbm.at[pl.ds(gbase, GD), :], ssem).wait()


_sc_msg = functools.partial(
    pl.kernel,
    out_type=jax.ShapeDtypeStruct((E, 128), _f32),
    mesh=_vmesh(),
    scratch_types=[
        pltpu.VMEM((CG, GD), _i32),
        pltpu.VMEM((CG, GD), _i32),
        pltpu.VMEM((GD, DA), _f32),
        pltpu.VMEM((GD, 128), _f32),
        pltpu.VMEM((GD, 128), _f32),
        pltpu.VMEM((GD, DA), _f32),
        pltpu.VMEM((GD, 128), _f32),
        pltpu.VMEM((GD, 128), _f32),
        pltpu.SemaphoreType.DMA,
        pltpu.SemaphoreType.DMA,
        pltpu.SemaphoreType.DMA,
    ],
)(_sc_msg_body)


def _sc_scat_body(r_hbm, dst_hbm, z_hbm, opart_hbm,
                  dsti, dsttA, dsttB, rowsA, rowsB, acc,
                  lsA, lsB, ssA, ssB):
    cid = lax.axis_index("c")
    sid = lax.axis_index("s")
    wid = cid * NS + sid
    pltpu.sync_copy(dst_hbm.at[wid], dsti)
    trap = jnp.full((16,), NR, _i32)

    def ld(g, rows, ls):
        return pltpu.make_async_copy(
            r_hbm.at[pl.ds(wid * EPW + g * G3, G3), :], rows, ls)

    def remap(g, dstt, base):
        for kk in range(G3 // 16):
            v = dsti[g, pl.ds(kk * 16, 16)] - base
            m = (v >= 0) & (v < NR)
            dstt[pl.ds(kk * 16, 16)] = jnp.where(m, v, trap)

    for k in range(NP):
        pltpu.sync_copy(z_hbm, acc.at[pl.ds(sid * HPT, HPT), :])

        @pl.when(sid == 0)
        def _():
            pltpu.sync_copy(z_hbm.at[pl.ds(0, HREM), :],
                            acc.at[pl.ds(NS * HPT, HREM), :])

        plsc.subcore_barrier()
        base = jnp.broadcast_to(jnp.int32(k * NR), (16,))
        ld(0, rowsA, lsA).start()

        def pair(p, c):
            gA = 2 * p
            gB = gA + 1

            @pl.when(p > 0)
            def _():
                pltpu.make_async_copy(rowsB, acc.at[dsttB], ssB).wait()

            @pl.when(gB < NG3)
            def _():
                ld(gB, rowsB, lsB).start()

            ld(gA, rowsA, lsA).wait()
            remap(gA, dsttA, base)
            pltpu.make_async_copy(rowsA, acc.at[dsttA], ssA).start(add=True)

            @pl.when(gB < NG3)
            def _():
                ld(gB, rowsB, lsB).wait()
                remap(gB, dsttB, base)
                pltpu.make_async_copy(rowsB, acc.at[dsttB], ssB).start(
                    add=True)

            @pl.when(p < (NG3 + 1) // 2 - 1)
            def _():
                pltpu.make_async_copy(rowsA, acc.at[dsttA], ssA).wait()
                ld(gA + 2, rowsA, lsA).start()
            return c

        lax.fori_loop(0, (NG3 + 1) // 2, pair, 0)
        pltpu.make_async_copy(rowsA, acc.at[dsttA], ssA).wait()
        plsc.subcore_barrier()
        pltpu.sync_copy(acc.at[pl.ds(sid * HPT, HPT), :],
                        opart_hbm.at[k, cid, pl.ds(sid * HPT, HPT), :])

        @pl.when(sid == 0)
        def _():
            pltpu.sync_copy(acc.at[pl.ds(NS * HPT, HREM), :],
                            opart_hbm.at[k, cid, pl.ds(NS * HPT, HREM), :])

        plsc.subcore_barrier()


_sc_scat = functools.partial(
    pl.kernel,
    out_type=jax.ShapeDtypeStruct((NP, NC, RP, 128), _f32),
    mesh=_vmesh(),
    scratch_types=[
        pltpu.VMEM((NG3, G3), _i32),
        pltpu.VMEM((G3,), _i32),
        pltpu.VMEM((G3,), _i32),
        pltpu.VMEM((G3, 128), _f32),
        pltpu.VMEM((G3, 128), _f32),
        pltpu.VMEM_SHARED((RP, 128), _f32),
        pltpu.SemaphoreType.DMA,
        pltpu.SemaphoreType.DMA,
        pltpu.SemaphoreType.DMA,
        pltpu.SemaphoreType.DMA,
    ],
)(_sc_scat_body)


def _segment_sum_parts(rows, dsts, z8):
    p = _sc_scat(rows, dsts, z8)
    s0 = jnp.concatenate(
        [p[0, 0, :NR], p[1, 0, :NR], p[2, 0, :N - 2 * NR]], axis=0)
    s1 = jnp.concatenate(
        [p[0, 1, :NR], p[1, 1, :NR], p[2, 1, :N - 2 * NR]], axis=0)
    return s0, s1


_BM = 2000


def _tc_l1_body(x_ref, w_ref, p_ref, h_ref, al_ref):
    xb = x_ref[...]
    h_ref[...] = jnp.dot(xb, w_ref[...], preferred_element_type=_f32)
    al_ref[...] = jnp.dot(xb, p_ref[...], preferred_element_type=_f32)


def _tc_layer1(x, Wcat, Pc):
    return pl.pallas_call(
        _tc_l1_body,
        grid=(N // _BM,),
        in_specs=[
            pl.BlockSpec((_BM, D), lambda i: (i, 0)),
            pl.BlockSpec((D, DA), lambda i: (0, 0)),
            pl.BlockSpec((D, 128), lambda i: (0, 0)),
        ],
        out_specs=[
            pl.BlockSpec((_BM, DA), lambda i: (i, 0)),
            pl.BlockSpec((_BM, 128), lambda i: (i, 0)),
        ],
        out_shape=[
            jax.ShapeDtypeStruct((N, DA), _f32),
            jax.ShapeDtypeStruct((N, 128), _f32),
        ],
    )(x, Wcat, Pc)


def _tc_l2_body(p0_ref, p1_ref, b_ref, w_ref, pc_ref, h_ref, al_ref):
    xb = jnp.maximum(p0_ref[...] + p1_ref[...] + b_ref[...], 0.0)
    h_ref[...] = jnp.dot(xb, w_ref[...], preferred_element_type=_f32)
    al_ref[...] = jnp.dot(xb, pc_ref[...], preferred_element_type=_f32)


def _tc_layer2(p0, p1, b, Wcat, Pc):
    return pl.pallas_call(
        _tc_l2_body,
        grid=(N // _BM,),
        in_specs=[
            pl.BlockSpec((_BM, D), lambda i: (i, 0)),
            pl.BlockSpec((_BM, D), lambda i: (i, 0)),
            pl.BlockSpec((1, D), lambda i: (0, 0)),
            pl.BlockSpec((D, DA), lambda i: (0, 0)),
            pl.BlockSpec((D, 128), lambda i: (0, 0)),
        ],
        out_specs=[
            pl.BlockSpec((_BM, DA), lambda i: (i, 0)),
            pl.BlockSpec((_BM, 128), lambda i: (i, 0)),
        ],
        out_shape=[
            jax.ShapeDtypeStruct((N, DA), _f32),
            jax.ShapeDtypeStruct((N, 128), _f32),
        ],
    )(p0, p1, b.reshape(1, D), Wcat, Pc)


def _tc_tdst_body(d0_ref, d1_ref, al_ref, t_ref):
    den = d0_ref[...] + d1_ref[...]
    rden = 1.0 / (8.0 * den[:, 0:8])
    adst = al_ref[...][:, 8:16]
    t_ref[...] = jnp.concatenate(
        [adst, rden, jnp.zeros((d0_ref.shape[0], 112), _f32)], axis=1)


def _tc_tdst(d0, d1, al):
    return pl.pallas_call(
        _tc_tdst_body,
        grid=(N // _BM,),
        in_specs=[
            pl.BlockSpec((_BM, 16), lambda i: (i, 0)),
            pl.BlockSpec((_BM, 16), lambda i: (i, 0)),
            pl.BlockSpec((_BM, 128), lambda i: (i, 0)),
        ],
        out_specs=pl.BlockSpec((_BM, 128), lambda i: (i, 0)),
        out_shape=jax.ShapeDtypeStruct((N, 128), _f32),
    )(d0, d1, al)


def _tc_final_body(p0_ref, p1_ref, b_ref, o_ref):
    o_ref[...] = p0_ref[...] + p1_ref[...] + b_ref[...]


def _tc_final(p0, p1, b):
    return pl.pallas_call(
        _tc_final_body,
        grid=(N // _BM,),
        in_specs=[
            pl.BlockSpec((_BM, D), lambda i: (i, 0)),
            pl.BlockSpec((_BM, D), lambda i: (i, 0)),
            pl.BlockSpec((1, D), lambda i: (0, 0)),
        ],
        out_specs=pl.BlockSpec((_BM, D), lambda i: (i, 0)),
        out_shape=jax.ShapeDtypeStruct((N, D), _f32),
    )(p0, p1, b.reshape(1, D))


def _pack_weights(W, a_src, a_dst):
    Wr = W.reshape(D, H, DH)
    ps = jnp.einsum("ihd,hd->ih", Wr, a_src)
    pd = jnp.einsum("ihd,hd->ih", Wr, a_dst)
    Wcat = jnp.concatenate([W, ps, jnp.zeros((D, 120), _f32)], axis=1)
    Pc = jnp.concatenate([ps, pd, jnp.zeros((D, 112), _f32)], axis=1)
    return Wcat, Pc


def _gat_layer(h, al, srca, dsta, srcd, dstd, dsts, z8):
    w = _sc_wraw(al, srca, dsta)
    dp = _sc_scat1(w, dsts, z8)
    d0 = dp[0, :N // 8].reshape(N, 16)
    d1 = dp[1, :N // 8].reshape(N, 16)
    t = _tc_tdst(d0, d1, al)
    m = _sc_msg(h, t, srcd, dstd)
    return _segment_sum_parts(m, dsts, z8)


def kernel(x, edge_index, W1, a_src1, a_dst1, b1, W2, a_src2, a_dst2, b2):
    src = edge_index[0].astype(_i32)
    dst = edge_index[1].astype(_i32)
    srca = src.reshape(NW, NGA, GA)
    dsta = dst.reshape(NW, NGA, GA)
    srcd = src.reshape(NW, NCH, CG, GD)
    dstd = dst.reshape(NW, NCH, CG, GD)
    dsts = dst.reshape(NW, NG3, G3)
    z8 = jnp.zeros((HPT, 128), _f32)
    Wcat1, Pc1 = _pack_weights(W1, a_src1, a_dst1)
    Wcat2, Pc2 = _pack_weights(W2, a_src2, a_dst2)

    h1, al1 = _tc_layer1(x, Wcat1, Pc1)
    o10, o11 = _gat_layer(h1, al1, srca, dsta, srcd, dstd, dsts, z8)

    h2, al2 = _tc_layer2(o10, o11, b1, Wcat2, Pc2)
    o20, o21 = _gat_layer(h2, al2, srca, dsta, srcd, dstd, dsts, z8)

    return _tc_final(o20, o21, b2)


# speedup vs baseline: 28.8886x; 1.2149x over previous
import functools

import jax
import jax.numpy as jnp
from jax import lax
from jax.experimental import pallas as pl
from jax.experimental.pallas import tpu as pltpu
from jax.experimental.pallas import tpu_sc as plsc

N = 10000
E = 320000
D = 128
H = 8
DH = 128
DA = H * DH + 128

NC = 2
NS = 16
NW = NC * NS
EPW = E // NW

GA = 80
NGA = EPW // GA
GD = 40
NGD = EPW // GD
G3 = 80
NG3 = EPW // G3

NR = 4664
NP = 3
RP = NR + 8
HPT = 288
HREM = RP - NS * HPT

_f32 = jnp.float32
_i32 = jnp.int32


def _vmesh():
    return plsc.VectorSubcoreMesh(core_axis_name="c", subcore_axis_name="s")


def _gather16(v, idx):
    return lax.gather(
        v,
        idx[:, None],
        dimension_numbers=lax.GatherDimensionNumbers(
            offset_dims=(), collapsed_slice_dims=(0,), start_index_map=(0,)),
        slice_sizes=(1,),
        mode=lax.GatherScatterMode.PROMISE_IN_BOUNDS,
    )


def _rot8(v):
    idx = lax.rem(lax.iota(_i32, 16) + 8, jnp.full((16,), 16, _i32))
    return _gather16(v, idx)


def _splat(v, lane):
    return _gather16(v, jnp.full((16,), lane, _i32))


RP1 = 1280
HPT1 = RP1 // NS


def _den_compute(g, dsti, gs, gd, wv, dstt):
    def blk(kb, c):
        lane = lax.iota(_i32, 16)
        lmask = jnp.minimum(jnp.maximum(8 - lane, 0), 1).astype(_f32)
        d16 = dsti[g, pl.ds(kb * 16, 16)]
        dstt[pl.ds(kb * 16, 16)] = lax.shift_right_logical(d16, 3)
        for ii in range(16):
            i = kb * 16 + ii
            slot = lax.bitwise_and(_splat(d16, ii),
                                   jnp.full((16,), 7, _i32))
            s = gs[i, pl.ds(0, 16)] + _rot8(gd[i, pl.ds(0, 16)])
            w = jnp.exp(jnp.maximum(s, 0.2 * s)) * lmask
            for jj in range(8):
                eq = (1 - jnp.minimum(jnp.abs(slot - jj), 1)).astype(_f32)
                wv[i, pl.ds(jj * 16, 16)] = w * eq
        return c
    lax.fori_loop(0, GA // 16, blk, 0)


def _sc_den_body(al_hbm, src_hbm, dst_hbm, z_hbm, dpart_hbm,
                 srci, dsti, gsA, gdA, wvA, dsttA, gsB, gdB, wvB, dsttB,
                 acc, lsA, lsB, ssA, ssB):
    cid = lax.axis_index("c")
    sid = lax.axis_index("s")
    wid = cid * NS + sid
    pltpu.sync_copy(src_hbm.at[wid], srci)
    pltpu.sync_copy(dst_hbm.at[wid], dsti)
    pltpu.sync_copy(z_hbm.at[pl.ds(0, HPT1), :],
                    acc.at[pl.ds(sid * HPT1, HPT1), :])
    plsc.subcore_barrier()

    def start_ld(g, gs, gd, ls):
        pltpu.make_async_copy(al_hbm.at[srci.at[g]], gs, ls).start()
        pltpu.make_async_copy(al_hbm.at[dsti.at[g]], gd, ls).start()

    def wait_ld(g, gs, gd, ls):
        pltpu.make_async_copy(al_hbm.at[srci.at[g]], gs, ls).wait()
        pltpu.make_async_copy(al_hbm.at[dsti.at[g]], gd, ls).wait()

    start_ld(0, gsA, gdA, lsA)

    def pair(p, c):
        gA = 2 * p
        gB = gA + 1

        @pl.when(gB < NGA)
        def _():
            start_ld(gB, gsB, gdB, lsB)

        wait_ld(gA, gsA, gdA, lsA)

        @pl.when(p > 0)
        def _():
            pltpu.make_async_copy(wvA, acc.at[dsttA], ssA).wait()

        _den_compute(gA, dsti, gsA, gdA, wvA, dsttA)
        pltpu.make_async_copy(wvA, acc.at[dsttA], ssA).start(add=True)

        @pl.when(p < (NGA + 1) // 2 - 1)
        def _():
            start_ld(gA + 2, gsA, gdA, lsA)

        @pl.when(gB < NGA)
        def _():
            wait_ld(gB, gsB, gdB, lsB)

            @pl.when(p > 0)
            def _():
                pltpu.make_async_copy(wvB, acc.at[dsttB], ssB).wait()

            _den_compute(gB, dsti, gsB, gdB, wvB, dsttB)
            pltpu.make_async_copy(wvB, acc.at[dsttB], ssB).start(add=True)
        return c

    lax.fori_loop(0, (NGA + 1) // 2, pair, 0)
    pltpu.make_async_copy(wvA, acc.at[dsttA], ssA).wait()
    pltpu.make_async_copy(wvB, acc.at[dsttB], ssB).wait()
    plsc.subcore_barrier()
    pltpu.sync_copy(acc.at[pl.ds(sid * HPT1, HPT1), :],
                    dpart_hbm.at[cid, pl.ds(sid * HPT1, HPT1), :])


_sc_den = functools.partial(
    pl.kernel,
    out_type=jax.ShapeDtypeStruct((NC, RP1, 128), _f32),
    mesh=_vmesh(),
    scratch_types=[
        pltpu.VMEM((NGA, GA), _i32),
        pltpu.VMEM((NGA, GA), _i32),
        pltpu.VMEM((GA, 128), _f32),
        pltpu.VMEM((GA, 128), _f32),
        pltpu.VMEM((GA, 128), _f32),
        pltpu.VMEM((GA,), _i32),
        pltpu.VMEM((GA, 128), _f32),
        pltpu.VMEM((GA, 128), _f32),
        pltpu.VMEM((GA, 128), _f32),
        pltpu.VMEM((GA,), _i32),
        pltpu.VMEM_SHARED((RP1, 128), _f32),
        pltpu.SemaphoreType.DMA,
        pltpu.SemaphoreType.DMA,
        pltpu.SemaphoreType.DMA,
        pltpu.SemaphoreType.DMA,
    ],
)(_sc_den_body)


NCH = 5
CG = NGD // NCH


def _msg_compute(hg, tg, msg):
    def ed(i, c):
        s = hg[i, pl.ds(H * DH, 16)]
        t = tg[i, pl.ds(0, 16)]
        u = s + t
        wr = jnp.exp(jnp.maximum(u, 0.2 * u)) * _rot8(u)
        accs = [None] * 8
        for h in range(H):
            sp = _splat(wr, h)
            for j in range(8):
                tt = sp * hg[i, pl.ds(h * 128 + j * 16, 16)]
                accs[j] = tt if h == 0 else accs[j] + tt
        for j in range(8):
            msg[i, pl.ds(j * 16, 16)] = accs[j]
        return c
    lax.fori_loop(0, GD, ed, 0)


def _sc_msg_body(h_hbm, t_hbm, src_hbm, dst_hbm, m_hbm,
                 srci, dsti, hgA, tgA, msgA, hgB, tgB, msgB,
                 lsA, lsB, ssem):
    cid = lax.axis_index("c")
    sid = lax.axis_index("s")
    wid = cid * NS + sid

    def start_ld(g, hg, tg, ls):
        pltpu.make_async_copy(h_hbm.at[srci.at[g]], hg, ls).start()
        pltpu.make_async_copy(t_hbm.at[dsti.at[g]], tg, ls).start()

    def wait_ld(g, hg, tg, ls):
        pltpu.make_async_copy(h_hbm.at[srci.at[g]], hg, ls).wait()
        pltpu.make_async_copy(t_hbm.at[dsti.at[g]], tg, ls).wait()

    for o in range(NCH):
        pltpu.sync_copy(src_hbm.at[wid, o], srci)
        pltpu.sync_copy(dst_hbm.at[wid, o], dsti)
        gbase = wid * EPW + o * CG * GD
        start_ld(0, hgA, tgA, lsA)

        def pair(p, c):
            g0 = 2 * p
            g1 = g0 + 1
            start_ld(g1, hgB, tgB, lsB)

            @pl.when(p > 0)
            def _():
                pltpu.make_async_copy(
                    msgA, m_hbm.at[pl.ds(gbase, GD), :], ssem).wait()
                pltpu.make_async_copy(
                    msgB, m_hbm.at[pl.ds(gbase, GD), :], ssem).wait()

            wait_ld(g0, hgA, tgA, lsA)
            _msg_compute(hgA, tgA, msgA)
            pltpu.make_async_copy(
                msgA, m_hbm.at[pl.ds(gbase + g0 * GD, GD), :], ssem).start()

            @pl.when(p < CG // 2 - 1)
            def _():
                start_ld(g0 + 2, hgA, tgA, lsA)

            wait_ld(g1, hgB, tgB, lsB)
            _msg_compute(hgB, tgB, msgB)
            pltpu.make_async_copy(
                msgB, m_hbm.at[pl.ds(gbase + g1 * GD, GD), :], ssem).start()
            return c

        lax.fori_loop(0, CG // 2, pair, 0)
        pltpu.make_async_copy(msgA, m_hbm.at[pl.ds(gbase, GD), :], ssem).wait()
        pltpu.make_async_copy(msgB, m_hbm.at[pl.ds(gbase, GD), :], ssem).wait()


_sc_msg = functools.partial(
    pl.kernel,
    out_type=jax.ShapeDtypeStruct((E, 128), _f32),
    mesh=_vmesh(),
    scratch_types=[
        pltpu.VMEM((CG, GD), _i32),
        pltpu.VMEM((CG, GD), _i32),
        pltpu.VMEM((GD, DA), _f32),
        pltpu.VMEM((GD, 128), _f32),
        pltpu.VMEM((GD, 128), _f32),
        pltpu.VMEM((GD, DA), _f32),
        pltpu.VMEM((GD, 128), _f32),
        pltpu.VMEM((GD, 128), _f32),
        pltpu.SemaphoreType.DMA,
        pltpu.SemaphoreType.DMA,
        pltpu.SemaphoreType.DMA,
    ],
)(_sc_msg_body)


def _sc_scat_body(r_hbm, dst_hbm, z_hbm, opart_hbm,
                  dsti, dsttA, dsttB, rowsA, rowsB, acc,
                  lsA, lsB, ssA, ssB):
    cid = lax.axis_index("c")
    sid = lax.axis_index("s")
    wid = cid * NS + sid
    pltpu.sync_copy(dst_hbm.at[wid], dsti)
    trap = jnp.full((16,), NR, _i32)

    def ld(g, rows, ls):
        return pltpu.make_async_copy(
            r_hbm.at[pl.ds(wid * EPW + g * G3, G3), :], rows, ls)

    def remap(g, dstt, base):
        for kk in range(G3 // 16):
            v = dsti[g, pl.ds(kk * 16, 16)] - base
            m = (v >= 0) & (v < NR)
            dstt[pl.ds(kk * 16, 16)] = jnp.where(m, v, trap)

    for k in range(NP):
        pltpu.sync_copy(z_hbm, acc.at[pl.ds(sid * HPT, HPT), :])

        @pl.when(sid == 0)
        def _():
            pltpu.sync_copy(z_hbm.at[pl.ds(0, HREM), :],
                            acc.at[pl.ds(NS * HPT, HREM), :])

        plsc.subcore_barrier()
        base = jnp.broadcast_to(jnp.int32(k * NR), (16,))
        ld(0, rowsA, lsA).start()

        def pair(p, c):
            gA = 2 * p
            gB = gA + 1

            @pl.when(p > 0)
            def _():
                pltpu.make_async_copy(rowsB, acc.at[dsttB], ssB).wait()

            @pl.when(gB < NG3)
            def _():
                ld(gB, rowsB, lsB).start()

            ld(gA, rowsA, lsA).wait()
            remap(gA, dsttA, base)
            pltpu.make_async_copy(rowsA, acc.at[dsttA], ssA).start(add=True)

            @pl.when(gB < NG3)
            def _():
                ld(gB, rowsB, lsB).wait()
                remap(gB, dsttB, base)
                pltpu.make_async_copy(rowsB, acc.at[dsttB], ssB).start(
                    add=True)

            @pl.when(p < (NG3 + 1) // 2 - 1)
            def _():
                pltpu.make_async_copy(rowsA, acc.at[dsttA], ssA).wait()
                ld(gA + 2, rowsA, lsA).start()
            return c

        lax.fori_loop(0, (NG3 + 1) // 2, pair, 0)
        pltpu.make_async_copy(rowsA, acc.at[dsttA], ssA).wait()
        plsc.subcore_barrier()
        pltpu.sync_copy(acc.at[pl.ds(sid * HPT, HPT), :],
                        opart_hbm.at[k, cid, pl.ds(sid * HPT, HPT), :])

        @pl.when(sid == 0)
        def _():
            pltpu.sync_copy(acc.at[pl.ds(NS * HPT, HREM), :],
                            opart_hbm.at[k, cid, pl.ds(NS * HPT, HREM), :])

        plsc.subcore_barrier()


_sc_scat = functools.partial(
    pl.kernel,
    out_type=jax.ShapeDtypeStruct((NP, NC, RP, 128), _f32),
    mesh=_vmesh(),
    scratch_types=[
        pltpu.VMEM((NG3, G3), _i32),
        pltpu.VMEM((G3,), _i32),
        pltpu.VMEM((G3,), _i32),
        pltpu.VMEM((G3, 128), _f32),
        pltpu.VMEM((G3, 128), _f32),
        pltpu.VMEM_SHARED((RP, 128), _f32),
        pltpu.SemaphoreType.DMA,
        pltpu.SemaphoreType.DMA,
        pltpu.SemaphoreType.DMA,
        pltpu.SemaphoreType.DMA,
    ],
)(_sc_scat_body)


def _segment_sum_parts(rows, dsts, z8):
    p = _sc_scat(rows, dsts, z8)
    s0 = jnp.concatenate(
        [p[0, 0, :NR], p[1, 0, :NR], p[2, 0, :N - 2 * NR]], axis=0)
    s1 = jnp.concatenate(
        [p[0, 1, :NR], p[1, 1, :NR], p[2, 1, :N - 2 * NR]], axis=0)
    return s0, s1


_BM = 2000


def _tc_l1_body(x_ref, w_ref, p_ref, h_ref, al_ref):
    xb = x_ref[...]
    h_ref[...] = jnp.dot(xb, w_ref[...], preferred_element_type=_f32)
    al_ref[...] = jnp.dot(xb, p_ref[...], preferred_element_type=_f32)


def _tc_layer1(x, Wcat, Pc):
    return pl.pallas_call(
        _tc_l1_body,
        grid=(N // _BM,),
        in_specs=[
            pl.BlockSpec((_BM, D), lambda i: (i, 0)),
            pl.BlockSpec((D, DA), lambda i: (0, 0)),
            pl.BlockSpec((D, 128), lambda i: (0, 0)),
        ],
        out_specs=[
            pl.BlockSpec((_BM, DA), lambda i: (i, 0)),
            pl.BlockSpec((_BM, 128), lambda i: (i, 0)),
        ],
        out_shape=[
            jax.ShapeDtypeStruct((N, DA), _f32),
            jax.ShapeDtypeStruct((N, 128), _f32),
        ],
    )(x, Wcat, Pc)


def _tc_l2_body(p0_ref, p1_ref, b_ref, w_ref, pc_ref, h_ref, al_ref):
    xb = jnp.maximum(p0_ref[...] + p1_ref[...] + b_ref[...], 0.0)
    h_ref[...] = jnp.dot(xb, w_ref[...], preferred_element_type=_f32)
    al_ref[...] = jnp.dot(xb, pc_ref[...], preferred_element_type=_f32)


def _tc_layer2(p0, p1, b, Wcat, Pc):
    return pl.pallas_call(
        _tc_l2_body,
        grid=(N // _BM,),
        in_specs=[
            pl.BlockSpec((_BM, D), lambda i: (i, 0)),
            pl.BlockSpec((_BM, D), lambda i: (i, 0)),
            pl.BlockSpec((1, D), lambda i: (0, 0)),
            pl.BlockSpec((D, DA), lambda i: (0, 0)),
            pl.BlockSpec((D, 128), lambda i: (0, 0)),
        ],
        out_specs=[
            pl.BlockSpec((_BM, DA), lambda i: (i, 0)),
            pl.BlockSpec((_BM, 128), lambda i: (i, 0)),
        ],
        out_shape=[
            jax.ShapeDtypeStruct((N, DA), _f32),
            jax.ShapeDtypeStruct((N, 128), _f32),
        ],
    )(p0, p1, b.reshape(1, D), Wcat, Pc)


def _tc_tdst_body(d0_ref, d1_ref, al_ref, t_ref):
    den = d0_ref[...] + d1_ref[...]
    rden = 1.0 / (8.0 * den[:, 0:8])
    adst = al_ref[...][:, 8:16]
    t_ref[...] = jnp.concatenate(
        [adst, rden, jnp.zeros((d0_ref.shape[0], 112), _f32)], axis=1)


def _tc_tdst(d0, d1, al):
    return pl.pallas_call(
        _tc_tdst_body,
        grid=(N // _BM,),
        in_specs=[
            pl.BlockSpec((_BM, 16), lambda i: (i, 0)),
            pl.BlockSpec((_BM, 16), lambda i: (i, 0)),
            pl.BlockSpec((_BM, 128), lambda i: (i, 0)),
        ],
        out_specs=pl.BlockSpec((_BM, 128), lambda i: (i, 0)),
        out_shape=jax.ShapeDtypeStruct((N, 128), _f32),
    )(d0, d1, al)


def _tc_final_body(p0_ref, p1_ref, b_ref, o_ref):
    o_ref[...] = p0_ref[...] + p1_ref[...] + b_ref[...]


def _tc_final(p0, p1, b):
    return pl.pallas_call(
        _tc_final_body,
        grid=(N // _BM,),
        in_specs=[
            pl.BlockSpec((_BM, D), lambda i: (i, 0)),
            pl.BlockSpec((_BM, D), lambda i: (i, 0)),
            pl.BlockSpec((1, D), lambda i: (0, 0)),
        ],
        out_specs=pl.BlockSpec((_BM, D), lambda i: (i, 0)),
        out_shape=jax.ShapeDtypeStruct((N, D), _f32),
    )(p0, p1, b.reshape(1, D))


def _pack_weights(W, a_src, a_dst):
    Wr = W.reshape(D, H, DH)
    ps = jnp.einsum("ihd,hd->ih", Wr, a_src)
    pd = jnp.einsum("ihd,hd->ih", Wr, a_dst)
    Wcat = jnp.concatenate([W, ps, jnp.zeros((D, 120), _f32)], axis=1)
    Pc = jnp.concatenate([ps, pd, jnp.zeros((D, 112), _f32)], axis=1)
    return Wcat, Pc


def _gat_layer(h, al, srca, dsta, srcd, dstd, dsts, z8):
    dp = _sc_den(al, srca, dsta, z8)
    d0 = dp[0, :N // 8].reshape(N, 16)
    d1 = dp[1, :N // 8].reshape(N, 16)
    t = _tc_tdst(d0, d1, al)
    m = _sc_msg(h, t, srcd, dstd)
    return _segment_sum_parts(m, dsts, z8)


def kernel(x, edge_index, W1, a_src1, a_dst1, b1, W2, a_src2, a_dst2, b2):
    src = edge_index[0].astype(_i32)
    dst = edge_index[1].astype(_i32)
    srca = src.reshape(NW, NGA, GA)
    dsta = dst.reshape(NW, NGA, GA)
    srcd = src.reshape(NW, NCH, CG, GD)
    dstd = dst.reshape(NW, NCH, CG, GD)
    dsts = dst.reshape(NW, NG3, G3)
    z8 = jnp.zeros((HPT, 128), _f32)
    Wcat1, Pc1 = _pack_weights(W1, a_src1, a_dst1)
    Wcat2, Pc2 = _pack_weights(W2, a_src2, a_dst2)

    h1, al1 = _tc_layer1(x, Wcat1, Pc1)
    o10, o11 = _gat_layer(h1, al1, srca, dsta, srcd, dstd, dsts, z8)

    h2, al2 = _tc_layer2(o10, o11, b1, Wcat2, Pc2)
    o20, o21 = _gat_layer(h2, al2, srca, dsta, srcd, dstd, dsts, z8)

    return _tc_final(o20, o21, b2)
